# baseline (speedup 1.0000x reference)
import jax
import jax.numpy as jnp
from jax.experimental import pallas as pl
from jax.experimental.pallas import tpu as pltpu
from jax.experimental.pallas import tpu_sc as plsc

T_BLK = 2048
S_BLK = 512


def _argmin_kernel(src_ref, tar_ref, ssq_ref, tsq_ref, out_ref):
    t = tar_ref[...]
    tsq = tsq_ref[...]
    n_src = src_ref.shape[0]

    def tile_argmin(c):
        s = src_ref[pl.ds(c * S_BLK, S_BLK), :]
        mm2 = jax.lax.dot_general(
            s, t, (((1,), (0,)), ((), ())),
            preferred_element_type=jnp.float32)
        ssq = ssq_ref[pl.ds(c * S_BLK, S_BLK), :]
        d2 = (tsq + mm2) + ssq

        h = S_BLK // 2
        lo, hi = d2[:h], d2[h:]
        lt = hi < lo
        val = jnp.minimum(lo, hi)
        row = jax.lax.broadcasted_iota(jnp.int32, (h, T_BLK), 0)
        idx = jnp.where(lt, row + h, row)
        while h > 1:
            h //= 2
            vlo, vhi = val[:h], val[h:]
            ilo, ihi = idx[:h], idx[h:]
            lt = vhi < vlo
            val = jnp.minimum(vlo, vhi)
            idx = jnp.where(lt, ihi, ilo)
        return val, idx

    bv, bi = tile_argmin(jnp.int32(0))
    bc = jnp.zeros((1, T_BLK), jnp.int32)

    def body(c, carry):
        bv, bi, bc = carry
        val, idx = tile_argmin(c)
        mask = val < bv
        bv = jnp.where(mask, val, bv)
        bi = jnp.where(mask, idx, bi)
        bc = jnp.where(mask, c, bc)
        return bv, bi, bc

    bv, bi, bc = jax.lax.fori_loop(1, n_src // S_BLK, body, (bv, bi, bc))
    out_ref[...] = (bc * S_BLK + bi).reshape(1, 1, T_BLK)


def _nn_indices_pallas(src_V, tar_V):
    n_src = src_V.shape[0]
    n_tar = tar_V.shape[0]
    tsq = jnp.sum(tar_V * tar_V, axis=1).reshape(1, n_tar)
    ssq = jnp.sum(src_V * src_V, axis=1).reshape(n_src, 1)
    tar_bf = tar_V.T.astype(jnp.bfloat16)
    src_bf = (-2.0 * src_V).astype(jnp.bfloat16)
    n_blk = n_tar // T_BLK
    idx = pl.pallas_call(
        _argmin_kernel,
        grid=(n_blk,),
        in_specs=[
            pl.BlockSpec((n_src, 3), lambda i: (0, 0)),
            pl.BlockSpec((3, T_BLK), lambda i: (0, i)),
            pl.BlockSpec((n_src, 1), lambda i: (0, 0)),
            pl.BlockSpec((1, T_BLK), lambda i: (0, i)),
        ],
        out_specs=pl.BlockSpec((1, 1, T_BLK), lambda i: (i, 0, 0)),
        out_shape=jax.ShapeDtypeStruct((n_blk, 1, T_BLK), jnp.int32),
    )(src_bf, tar_bf, ssq, tsq)
    return idx.reshape(n_tar)


_SC_UNITS = 32
_SC_LANES = 16


_SC_WIN = 128


def _sc_gather_loss(src_pad, tar_pad, idx2d):
    n_tar = tar_pad.shape[0]
    per = n_tar // _SC_UNITS
    n_win = per // _SC_WIN

    mesh = plsc.VectorSubcoreMesh(core_axis_name="c", subcore_axis_name="s")

    @pl.kernel(
        out_type=jax.ShapeDtypeStruct((_SC_UNITS, _SC_LANES), jnp.float32),
        mesh=mesh,
        scratch_types=[
            pltpu.VMEM((1, per), jnp.int32),
            pltpu.VMEM((_SC_WIN, 128), jnp.float32),
            pltpu.VMEM((per, _SC_LANES), jnp.float32),
            pltpu.VMEM((1, _SC_LANES), jnp.float32),
            pltpu.SemaphoreType.DMA,
            pltpu.SemaphoreType.DMA,
        ])
    def k(src_hbm, tar_hbm, idx_hbm, o_hbm, idxv, gv, tv, acc, sem1, sem2):
        ci = jax.lax.axis_index("c")
        si = jax.lax.axis_index("s")
        unit = ci * (_SC_UNITS // 2) + si
        base = unit * per
        cp_i = pltpu.async_copy(idx_hbm.at[:, pl.ds(base, per)], idxv, sem1)
        cp_t = pltpu.async_copy(tar_hbm.at[pl.ds(base, per), :], tv, sem2)
        cp_i.wait()
        cp_t.wait()
        acc[...] = jnp.zeros((1, _SC_LANES), jnp.float32)

        @pl.loop(0, n_win)
        def _(w):
            pltpu.sync_copy(src_hbm.at[idxv.at[0, pl.ds(w * _SC_WIN, _SC_WIN)]],
                            gv)
            @pl.loop(0, _SC_WIN)
            def _(r):
                d = (gv[pl.ds(r, 1), : _SC_LANES]
                     - tv[pl.ds(w * _SC_WIN + r, 1), :])
                acc[...] += d * d

        pltpu.sync_copy(acc, o_hbm.at[pl.ds(unit, 1), :])

    return k(src_pad, tar_pad, idx2d)


def kernel(src_V, tar_V):
    idx = _nn_indices_pallas(src_V, tar_V)
    src_pad = jnp.pad(src_V, ((0, 0), (0, 128 - src_V.shape[1])))
    tar_pad = jnp.pad(tar_V, ((0, 0), (0, _SC_LANES - tar_V.shape[1])))
    partials = _sc_gather_loss(src_pad, tar_pad, idx.reshape(1, -1))
    return 0.5 * jnp.sum(partials)
